# Initial kernel scaffold; baseline (speedup 1.0000x reference)
import jax
import jax.numpy as jnp
from jax.experimental import pallas as pl


def kernel(x, raw, W, b):
    raise NotImplementedError("write your pallas kernel here")


# speedup vs baseline: 2.5965x; 2.5965x over previous
import functools

import jax
import jax.numpy as jnp
from jax.experimental import pallas as pl


BLK = 1024


def _body(x_ref, raw_ref, wf_ref, b_ref, out_ref, idx_ref, *, n_tiles, n_classes):
    xb = x_ref[:, :]
    rawv = raw_ref[:, :]
    sigs = jnp.where(rawv > 0.3, 1.0, jnp.where(rawv < -0.3, -1.0, 0.0))

    scores = jax.lax.dot_general(
        xb, sigs, (((1,), (1,)), ((), ())),
        preferred_element_type=jnp.float32)
    iota_t = jax.lax.broadcasted_iota(jnp.int32, scores.shape, 1)
    m = jnp.max(scores, axis=1, keepdims=True)
    idx = jnp.min(jnp.where(scores == m, iota_t, n_tiles), axis=1)

    alll = jax.lax.dot_general(
        xb, wf_ref[:, :], (((1,), (1,)), ((), ())),
        preferred_element_type=jnp.float32)
    lane = jax.lax.broadcasted_iota(jnp.int32, alll.shape, 1)
    masked = jnp.where((lane // n_classes) == idx[:, None], alll, 0.0)
    rowi = jax.lax.broadcasted_iota(jnp.int32, (n_tiles * n_classes, n_classes), 0)
    coli = jax.lax.broadcasted_iota(jnp.int32, (n_tiles * n_classes, n_classes), 1)
    fold = (rowi % n_classes == coli).astype(jnp.float32)
    logits = jax.lax.dot_general(
        masked, fold, (((1,), (0,)), ((), ())),
        preferred_element_type=jnp.float32)

    onehot = (iota_t == idx[:, None]).astype(jnp.float32)
    bsel = jax.lax.dot_general(
        onehot, b_ref[:, :], (((1,), (0,)), ((), ())),
        preferred_element_type=jnp.float32)

    out_ref[:, :] = logits + bsel
    idx_ref[0, 0, :] = idx


@jax.jit
def kernel(x, raw, W, b):
    n_tok, d_model = x.shape
    n_tiles, n_classes, _ = W.shape
    wf = W.reshape(n_tiles * n_classes, d_model)
    grid = n_tok // BLK

    logits, idx3 = pl.pallas_call(
        functools.partial(_body, n_tiles=n_tiles, n_classes=n_classes),
        grid=(grid,),
        in_specs=[
            pl.BlockSpec((BLK, d_model), lambda i: (i, 0)),
            pl.BlockSpec((n_tiles, d_model), lambda i: (0, 0)),
            pl.BlockSpec((n_tiles * n_classes, d_model), lambda i: (0, 0)),
            pl.BlockSpec((n_tiles, n_classes), lambda i: (0, 0)),
        ],
        out_specs=[
            pl.BlockSpec((BLK, n_classes), lambda i: (i, 0)),
            pl.BlockSpec((1, 1, BLK), lambda i: (i, 0, 0)),
        ],
        out_shape=[
            jax.ShapeDtypeStruct((n_tok, n_classes), jnp.float32),
            jax.ShapeDtypeStruct((grid, 1, BLK), jnp.int32),
        ],
    )(x, raw, wf, b)

    return logits, idx3.reshape(n_tok)
